# baseline (speedup 1.0000x reference)
import functools

import jax
import jax.numpy as jnp
from jax import lax
from jax.experimental import pallas as pl
from jax.experimental.pallas import tpu as pltpu
from jax.experimental.pallas import tpu_sc as plsc

_NC = 2
_NS = 16
_L = 16
_NW = _NC * _NS
_BT = 1024


def _imap(i):
    return (jnp.int32(i), jnp.int32(0))


def _dec_body(x_ref, o_ref):
    b, c = o_ref.shape
    o_ref[...] = x_ref[...].reshape(b, c, 2)[:, :, 0]


def _enc_body(x_ref, o_ref):
    v = x_ref[...]
    b, c = v.shape
    o_ref[...] = jnp.stack(
        [v, lax.shift_right_arithmetic(v, jnp.int32(31))], axis=-1
    ).reshape(b, 2 * c)


def _decode64(x2):
    n, c2 = x2.shape
    return pl.pallas_call(
        _dec_body,
        out_shape=jax.ShapeDtypeStruct((n, c2 // 2), jnp.int32),
        grid=(n // _BT,),
        in_specs=[pl.BlockSpec((_BT, c2), _imap)],
        out_specs=pl.BlockSpec((_BT, c2 // 2), _imap),
    )(x2)


def _encode64(x):
    n, c = x.shape
    return pl.pallas_call(
        _enc_body,
        out_shape=jax.ShapeDtypeStruct((n, 2 * c), jnp.int32),
        grid=(n // _BT,),
        in_specs=[pl.BlockSpec((_BT, c), _imap)],
        out_specs=pl.BlockSpec((_BT, 2 * c), _imap),
    )(x)


@functools.cache
def _sc_lookup_call(n: int, conv_words: int):
    n_per_w = n // _NW
    mesh = plsc.VectorSubcoreMesh(core_axis_name="c", subcore_axis_name="s")

    @functools.partial(
        pl.kernel,
        out_type=jax.ShapeDtypeStruct((n,), jnp.int32),
        mesh=mesh,
        scratch_types=[
            pltpu.VMEM((conv_words,), jnp.int32),
            pltpu.VMEM((n_per_w,), jnp.int32),
            pltpu.VMEM((n_per_w,), jnp.int32),
        ],
        compiler_params=pltpu.CompilerParams(needs_layout_passes=False),
    )
    def body(sp_hbm, conv_hbm, out_hbm, conv_v, sp_v, out_v):
        wid = lax.axis_index("s") * jnp.int32(_NC) + lax.axis_index("c")
        base = wid * jnp.int32(n_per_w)
        pltpu.sync_copy(conv_hbm, conv_v)
        pltpu.sync_copy(sp_hbm.at[pl.ds(base, n_per_w)], sp_v)

        @plsc.parallel_loop(jnp.int32(0), jnp.int32(n_per_w),
                            step=jnp.int32(_L), unroll=8)
        def _(off):
            idx = sp_v[pl.ds(off, _L)]
            out_v[pl.ds(off, _L)] = plsc.load_gather(conv_v, [idx])

        pltpu.sync_copy(out_v, out_hbm.at[pl.ds(base, n_per_w)])

    return body


def kernel(species, coordinates, conv_tensor):
    shape = species.shape
    n = species.size
    conv16 = (
        jnp.zeros((_L,), jnp.int32)
        .at[: conv_tensor.shape[0]]
        .set(conv_tensor.astype(jnp.int32))
    )
    if species.dtype.itemsize == 8:
        sp2 = lax.bitcast_convert_type(species, jnp.int32)
        lo = _decode64(sp2.reshape(shape[0], 2 * shape[1]))
    else:
        lo = species.astype(jnp.int32)
    out32 = _sc_lookup_call(n, _L)(lo.reshape(n), conv16)
    out32 = out32.reshape(shape)
    if conv_tensor.dtype.itemsize == 8:
        enc = _encode64(out32).reshape(shape + (2,))
        out = lax.bitcast_convert_type(enc, conv_tensor.dtype)
    else:
        out = out32.astype(conv_tensor.dtype)
    return out, coordinates
